# Initial kernel scaffold; baseline (speedup 1.0000x reference)
import jax
import jax.numpy as jnp
from jax.experimental import pallas as pl


def kernel(h, c, child_idx, W_f, b_f, W_iou, b_iou):
    raise NotImplementedError("write your pallas kernel here")


# speedup vs baseline: 2.8603x; 2.8603x over previous
import functools

import jax
import jax.numpy as jnp
from jax import lax
from jax.experimental import pallas as pl
from jax.experimental.pallas import tpu as pltpu
from jax.experimental.pallas import tpu_sc as plsc


def _sc_gather_body(NC, NCH, CH, HH, n_per_w,
                    h_hbm, c_hbm, idx0_hbm, idx1_hbm, out_h, out_c,
                    i0_v, i1_v, h0_v, h1_v, c0_v, c1_v, sem):
    wid = lax.axis_index("s") * NC + lax.axis_index("c")

    def chunk(k, carry):
        base = wid * n_per_w + k * CH
        pltpu.sync_copy(idx0_hbm.at[pl.ds(base, CH)], i0_v)
        pltpu.sync_copy(idx1_hbm.at[pl.ds(base, CH)], i1_v)
        cp0 = pltpu.async_copy(h_hbm.at[i0_v], h0_v, sem)
        cp1 = pltpu.async_copy(h_hbm.at[i1_v], h1_v, sem)
        cp2 = pltpu.async_copy(c_hbm.at[i0_v], c0_v, sem)
        cp3 = pltpu.async_copy(c_hbm.at[i1_v], c1_v, sem)
        cp0.wait()
        cp1.wait()
        cp2.wait()
        cp3.wait()
        pltpu.sync_copy(h0_v, out_h.at[pl.ds(base, CH), pl.ds(0, HH)])
        pltpu.sync_copy(h1_v, out_h.at[pl.ds(base, CH), pl.ds(HH, HH)])
        pltpu.sync_copy(c0_v, out_c.at[pl.ds(base, CH), pl.ds(0, HH)])
        pltpu.sync_copy(c1_v, out_c.at[pl.ds(base, CH), pl.ds(HH, HH)])
        return carry

    lax.fori_loop(0, NCH, chunk, 0)


def _make_sc_gather(n_pad, HH):
    info = plsc.get_sparse_core_info()
    NC, NS = info.num_cores, info.num_subcores
    NW = NC * NS
    assert n_pad % NW == 0
    n_per_w = n_pad // NW
    CH = 64
    assert n_per_w % CH == 0
    NCH = n_per_w // CH

    mesh = plsc.VectorSubcoreMesh(core_axis_name="c", subcore_axis_name="s")
    return functools.partial(
        pl.kernel,
        functools.partial(_sc_gather_body, NC, NCH, CH, HH, n_per_w),
        out_type=[jax.ShapeDtypeStruct((n_pad, 2 * HH), jnp.float32),
                  jax.ShapeDtypeStruct((n_pad, 2 * HH), jnp.float32)],
        mesh=mesh,
        scratch_types=[
            pltpu.VMEM((CH,), jnp.int32),
            pltpu.VMEM((CH,), jnp.int32),
            pltpu.VMEM((CH, HH), jnp.float32),
            pltpu.VMEM((CH, HH), jnp.float32),
            pltpu.VMEM((CH, HH), jnp.float32),
            pltpu.VMEM((CH, HH), jnp.float32),
            pltpu.SemaphoreType.DMA,
        ],
    )()


def _tc_body(HH, hcat_ref, cc_ref, wft_ref, bf_ref, wiout_ref, biou_ref,
             hnew_ref, cnew_ref):
    hcat = hcat_ref[...]
    f_pre = jnp.dot(hcat, wft_ref[...],
                    preferred_element_type=jnp.float32) + bf_ref[...]
    f = jax.nn.sigmoid(f_pre)
    cc = cc_ref[...]
    c_red = f[:, :HH] * cc[:, :HH] + f[:, HH:] * cc[:, HH:]
    iou = jnp.dot(hcat, wiout_ref[...],
                  preferred_element_type=jnp.float32) + biou_ref[...]
    i = jax.nn.sigmoid(iou[:, :HH])
    o = jax.nn.sigmoid(iou[:, HH:2 * HH])
    u = jnp.tanh(iou[:, 2 * HH:])
    c_new = i * u + c_red
    hnew_ref[...] = o * jnp.tanh(c_new)
    cnew_ref[...] = c_new


def _tc_compute(n, hcat, cc, wft, bf, wiout, biou, M=2000):
    twoH = hcat.shape[1]
    HH = twoH // 2
    grid = (n // M,)
    return pl.pallas_call(
        functools.partial(_tc_body, HH),
        grid=grid,
        in_specs=[
            pl.BlockSpec((M, twoH), lambda i: (i, 0)),
            pl.BlockSpec((M, twoH), lambda i: (i, 0)),
            pl.BlockSpec((twoH, twoH), lambda i: (0, 0)),
            pl.BlockSpec((1, twoH), lambda i: (0, 0)),
            pl.BlockSpec((twoH, 3 * HH), lambda i: (0, 0)),
            pl.BlockSpec((1, 3 * HH), lambda i: (0, 0)),
        ],
        out_specs=[
            pl.BlockSpec((M, HH), lambda i: (i, 0)),
            pl.BlockSpec((M, HH), lambda i: (i, 0)),
        ],
        out_shape=[
            jax.ShapeDtypeStruct((n, HH), jnp.float32),
            jax.ShapeDtypeStruct((n, HH), jnp.float32),
        ],
    )(hcat, cc, wft, bf, wiout, biou)


def kernel(h, c, child_idx, W_f, b_f, W_iou, b_iou):
    n, HH = h.shape
    NW = 32
    n_per_w = -(-n // (NW * 64)) * 64
    n_pad = n_per_w * NW

    ci = child_idx.astype(jnp.int32)
    pad = jnp.zeros((n_pad - n,), jnp.int32)
    idx0 = jnp.concatenate([ci[:, 0], pad])
    idx1 = jnp.concatenate([ci[:, 1], pad])

    sc_gather = _make_sc_gather(n_pad, HH)
    hcat, ccat = sc_gather(h, c, idx0, idx1)

    h_new, c_new = _tc_compute(
        n, hcat, ccat,
        W_f.T, b_f.reshape(1, -1), W_iou.T, b_iou.reshape(1, -1))
    return (h_new, c_new)


# speedup vs baseline: 3.5930x; 1.2562x over previous
import functools

import jax
import jax.numpy as jnp
from jax import lax
from jax.experimental import pallas as pl
from jax.experimental.pallas import tpu as pltpu
from jax.experimental.pallas import tpu_sc as plsc


def _sc_gather_body(NC, NCH, CH, HH, n_per_w, NBUF,
                    h_hbm, c_hbm, idx0_hbm, idx1_hbm, out_h, out_c,
                    i0_v, i1_v, h0_v, h1_v, c0_v, c1_v, sem0, sem1):
    wid = lax.axis_index("s") * NC + lax.axis_index("c")
    sems = (sem0, sem1)

    def gathers(b, k):
        base = wid * n_per_w + k * CH
        return (
            pltpu.make_async_copy(idx0_hbm.at[pl.ds(base, CH)], i0_v.at[b],
                                  sems[b]),
            pltpu.make_async_copy(idx1_hbm.at[pl.ds(base, CH)], i1_v.at[b],
                                  sems[b]),
            pltpu.make_async_copy(h_hbm.at[i0_v.at[b]], h0_v.at[b], sems[b]),
            pltpu.make_async_copy(h_hbm.at[i1_v.at[b]], h1_v.at[b], sems[b]),
            pltpu.make_async_copy(c_hbm.at[i0_v.at[b]], c0_v.at[b], sems[b]),
            pltpu.make_async_copy(c_hbm.at[i1_v.at[b]], c1_v.at[b], sems[b]),
        )

    def fire(b, k):
        cps = gathers(b, k)
        cps[0].start()
        cps[1].start()
        cps[0].wait()
        cps[1].wait()
        for cp in cps[2:]:
            cp.start()

    def drain_write(b, k):
        base = wid * n_per_w + k * CH
        for cp in gathers(b, k)[2:]:
            cp.wait()
        pltpu.sync_copy(h0_v.at[b], out_h.at[pl.ds(base, CH), pl.ds(0, HH)])
        pltpu.sync_copy(h1_v.at[b], out_h.at[pl.ds(base, CH), pl.ds(HH, HH)])
        pltpu.sync_copy(c0_v.at[b], out_c.at[pl.ds(base, CH), pl.ds(0, HH)])
        pltpu.sync_copy(c1_v.at[b], out_c.at[pl.ds(base, CH), pl.ds(HH, HH)])

    for b in range(NBUF):
        fire(b, b)

    def body(it, carry):
        g = it * NBUF
        for b in range(NBUF):
            k = g + b
            drain_write(b, k)

            @pl.when(k + NBUF < NCH)
            def _():
                fire(b, k + NBUF)
        return carry

    lax.fori_loop(0, NCH // NBUF, body, 0)


def _make_sc_gather(n_pad, HH):
    info = plsc.get_sparse_core_info()
    NC, NS = info.num_cores, info.num_subcores
    NW = NC * NS
    assert n_pad % NW == 0
    n_per_w = n_pad // NW
    CH = 40
    NBUF = 2
    assert n_per_w % CH == 0
    NCH = n_per_w // CH
    assert NCH % NBUF == 0

    mesh = plsc.VectorSubcoreMesh(core_axis_name="c", subcore_axis_name="s")
    return functools.partial(
        pl.kernel,
        functools.partial(_sc_gather_body, NC, NCH, CH, HH, n_per_w, NBUF),
        out_type=[jax.ShapeDtypeStruct((n_pad, 2 * HH), jnp.float32),
                  jax.ShapeDtypeStruct((n_pad, 2 * HH), jnp.float32)],
        mesh=mesh,
        scratch_types=[
            pltpu.VMEM((NBUF, CH), jnp.int32),
            pltpu.VMEM((NBUF, CH), jnp.int32),
            pltpu.VMEM((NBUF, CH, HH), jnp.float32),
            pltpu.VMEM((NBUF, CH, HH), jnp.float32),
            pltpu.VMEM((NBUF, CH, HH), jnp.float32),
            pltpu.VMEM((NBUF, CH, HH), jnp.float32),
            pltpu.SemaphoreType.DMA,
            pltpu.SemaphoreType.DMA,
        ],
    )()


def _tc_body(HH, hcat_ref, cc_ref, wft_ref, bf_ref, wiout_ref, biou_ref,
             hnew_ref, cnew_ref):
    hcat = hcat_ref[...]
    f_pre = jnp.dot(hcat, wft_ref[...],
                    preferred_element_type=jnp.float32) + bf_ref[...]
    f = jax.nn.sigmoid(f_pre)
    cc = cc_ref[...]
    c_red = f[:, :HH] * cc[:, :HH] + f[:, HH:] * cc[:, HH:]
    iou = jnp.dot(hcat, wiout_ref[...],
                  preferred_element_type=jnp.float32) + biou_ref[...]
    i = jax.nn.sigmoid(iou[:, :HH])
    o = jax.nn.sigmoid(iou[:, HH:2 * HH])
    u = jnp.tanh(iou[:, 2 * HH:])
    c_new = i * u + c_red
    hnew_ref[...] = o * jnp.tanh(c_new)
    cnew_ref[...] = c_new


def _tc_compute(n, hcat, cc, wft, bf, wiout, biou, M=2000):
    twoH = hcat.shape[1]
    HH = twoH // 2
    grid = (n // M,)
    return pl.pallas_call(
        functools.partial(_tc_body, HH),
        grid=grid,
        in_specs=[
            pl.BlockSpec((M, twoH), lambda i: (i, 0)),
            pl.BlockSpec((M, twoH), lambda i: (i, 0)),
            pl.BlockSpec((twoH, twoH), lambda i: (0, 0)),
            pl.BlockSpec((1, twoH), lambda i: (0, 0)),
            pl.BlockSpec((twoH, 3 * HH), lambda i: (0, 0)),
            pl.BlockSpec((1, 3 * HH), lambda i: (0, 0)),
        ],
        out_specs=[
            pl.BlockSpec((M, HH), lambda i: (i, 0)),
            pl.BlockSpec((M, HH), lambda i: (i, 0)),
        ],
        out_shape=[
            jax.ShapeDtypeStruct((n, HH), jnp.float32),
            jax.ShapeDtypeStruct((n, HH), jnp.float32),
        ],
    )(hcat, cc, wft, bf, wiout, biou)


def kernel(h, c, child_idx, W_f, b_f, W_iou, b_iou):
    n, HH = h.shape
    NW = 32
    n_per_w = -(-n // (NW * 80)) * 80
    n_pad = n_per_w * NW

    ci = child_idx.astype(jnp.int32)
    pad = jnp.zeros((n_pad - n,), jnp.int32)
    idx0 = jnp.concatenate([ci[:, 0], pad])
    idx1 = jnp.concatenate([ci[:, 1], pad])

    sc_gather = _make_sc_gather(n_pad, HH)
    hcat, ccat = sc_gather(h, c, idx0, idx1)

    h_new, c_new = _tc_compute(
        n, hcat, ccat,
        W_f.T, b_f.reshape(1, -1), W_iou.T, b_iou.reshape(1, -1))
    return (h_new, c_new)
